# Initial kernel scaffold; baseline (speedup 1.0000x reference)
import jax
import jax.numpy as jnp
from jax.experimental import pallas as pl


def kernel(words, labels, groups, masks, weight):
    raise NotImplementedError("write your pallas kernel here")


# speedup vs baseline: 8.5372x; 8.5372x over previous
import functools

import jax
import jax.numpy as jnp
from jax import lax
from jax.experimental import pallas as pl
from jax.experimental.pallas import tpu as pltpu
from jax.experimental.pallas import tpu_sc as plsc

D = 64
B, G, L = 1024, 26, 50
SEGS = B * G
NC, NS = 2, 16
NW = NC * NS
SEG_PER_CHUNK = 16
IDX_PER_CHUNK = SEG_PER_CHUNK * L
NSUB = 8
SUB = IDX_PER_CHUNK // NSUB
NCHUNK = SEGS // SEG_PER_CHUNK
CHUNK_PER_W = NCHUNK // NW
NLANE = 16
DV = D // NLANE


def _sc_body(weight_hbm, words_hbm, out_hbm, idx_v, rows_v, out_v, sem):
    wid = lax.axis_index("s") * NC + lax.axis_index("c")

    def chunk_body(c, carry):
        cg = wid * CHUNK_PER_W + c
        pltpu.sync_copy(words_hbm.at[cg], idx_v)
        handles = [
            pltpu.async_copy(
                weight_hbm.at[idx_v.at[j]],
                rows_v.at[pl.ds(j * SUB, SUB)],
                sem,
            )
            for j in range(NSUB)
        ]
        for h in handles:
            h.wait()

        def seg_body(s, carry2):
            base = s * L

            def l_body(l, accs):
                r = base + l
                return tuple(
                    accs[d] + rows_v[r, pl.ds(d * NLANE, NLANE)]
                    for d in range(DV)
                )

            accs = lax.fori_loop(
                0, L, l_body,
                tuple(jnp.zeros((NLANE,), jnp.float32) for _ in range(DV)),
            )
            for d in range(DV):
                out_v[s, pl.ds(d * NLANE, NLANE)] = accs[d] * (1.0 / L)
            return carry2

        lax.fori_loop(0, SEG_PER_CHUNK, seg_body, 0)
        pltpu.sync_copy(
            out_v, out_hbm.at[pl.ds(cg * SEG_PER_CHUNK, SEG_PER_CHUNK)]
        )
        return carry

    lax.fori_loop(0, CHUNK_PER_W, chunk_body, 0)


@jax.jit
def _gather_pool(weight, words_r):
    mesh = plsc.VectorSubcoreMesh(core_axis_name="c", subcore_axis_name="s")
    f = functools.partial(
        pl.kernel,
        mesh=mesh,
        out_type=jax.ShapeDtypeStruct((SEGS, D), jnp.float32),
        scratch_types=[
            pltpu.VMEM((NSUB, SUB), jnp.int32),
            pltpu.VMEM((IDX_PER_CHUNK, D), jnp.float32),
            pltpu.VMEM((SEG_PER_CHUNK, D), jnp.float32),
            pltpu.SemaphoreType.DMA,
        ],
        compiler_params=pltpu.CompilerParams(use_tc_tiling_on_sc=False),
    )(_sc_body)
    return f(weight, words_r)


def kernel(words, labels, groups, masks, weight):
    words_r = words.reshape(NCHUNK, NSUB, SUB).astype(jnp.int32)
    agg = _gather_pool(weight, words_r)
    return (agg.reshape(B, G, D), labels)
